# baseline (speedup 1.0000x reference)
import functools

import jax
import jax.numpy as jnp
from jax import lax
from jax.experimental import pallas as pl
from jax.experimental.pallas import tpu as pltpu
from jax.experimental.pallas import tpu_sc as plsc

N_NODES = 100000
N_EDGES = 1600000
HALF = 50000
TILES = 16
ACC_ROWS = HALF + TILES
EROWS_PER_TILE = 784
EROWS = TILES * EROWS_PER_TILE
E_PAD = EROWS * 128
CHUNK_ROWS = 14
N_CHUNKS = EROWS_PER_TILE // CHUNK_ROWS
ZROWS = 3200
ZTAIL = ACC_ROWS - 15 * ZROWS
OTAIL = HALF - 15 * ZROWS
FL = 40
FP = 48
DP = 8
BLK = 2000

_sc_mesh = plsc.VectorSubcoreMesh(core_axis_name="c", subcore_axis_name="s")
_sc_params = pltpu.CompilerParams(use_tc_tiling_on_sc=False)


def _sc_zero_acc(zeros_hbm, acc, sid):
    @pl.when(sid < TILES - 1)
    def _():
        pltpu.sync_copy(zeros_hbm, acc.at[pl.ds(sid * ZROWS, ZROWS)])

    @pl.when(sid == TILES - 1)
    def _():
        pltpu.sync_copy(zeros_hbm.at[pl.ds(0, ZTAIL)],
                        acc.at[pl.ds(15 * ZROWS, ZTAIL)])


def _sc_copy_out(acc, out_hbm, sid, node_base):
    @pl.when(sid < TILES - 1)
    def _():
        pltpu.sync_copy(acc.at[pl.ds(sid * ZROWS, ZROWS)],
                        out_hbm.at[pl.ds(node_base + sid * ZROWS, ZROWS)])

    @pl.when(sid == TILES - 1)
    def _():
        pltpu.sync_copy(acc.at[pl.ds(15 * ZROWS, OTAIL)],
                        out_hbm.at[pl.ds(node_base + 15 * ZROWS, OTAIL)])


def _remap_to_1d(stagev, out_refs, node_base, dummy, remap):
    for r in range(CHUNK_ROWS):
        for j in range(128 // 16):
            v = stagev[pl.ds(r * 128 + j * 16, 16)]
            if remap:
                local = v - node_base
                m = (local >= 0) & (local < HALF)
                v = jnp.where(m, local, dummy)
            out_refs[r][pl.ds(j * 16, 16)] = v


@functools.partial(
    pl.kernel,
    mesh=_sc_mesh,
    out_type=jax.ShapeDtypeStruct((N_NODES, FP), jnp.bfloat16),
    scratch_types=(
        [pltpu.VMEM((CHUNK_ROWS * 128,), jnp.int32)] * 2
        + [pltpu.VMEM((128,), jnp.int32)] * (2 * CHUNK_ROWS)
        + [
            pltpu.VMEM((CHUNK_ROWS * 128, FP), jnp.bfloat16),
            pltpu.VMEM_SHARED((ACC_ROWS, FP), jnp.bfloat16),
            pltpu.SemaphoreType.DMA,
            pltpu.SemaphoreType.DMA,
        ]
    ),
    compiler_params=_sc_params,
)
def _sc_scatter(src_hbm, dst_hbm, xs_hbm, zeros_hbm, out_hbm, *scratch):
    srcv, dstv = scratch[0], scratch[1]
    src_r = scratch[2:2 + CHUNK_ROWS]
    dst_r = scratch[2 + CHUNK_ROWS:2 + 2 * CHUNK_ROWS]
    rows, acc, gsem, ssem = scratch[2 + 2 * CHUNK_ROWS:]

    cid = lax.axis_index("c")
    sid = lax.axis_index("s")
    node_base = cid * HALF
    dummy = HALF + sid

    _sc_zero_acc(zeros_hbm, acc, sid)
    plsc.subcore_barrier()

    def chunk_body(c, carry):
        eb = (sid * EROWS_PER_TILE + c * CHUNK_ROWS) * 128
        pltpu.sync_copy(src_hbm.at[pl.ds(eb, CHUNK_ROWS * 128)], srcv)
        pltpu.sync_copy(dst_hbm.at[pl.ds(eb, CHUNK_ROWS * 128)], dstv)
        _remap_to_1d(srcv, src_r, node_base, dummy, remap=False)
        _remap_to_1d(dstv, dst_r, node_base, dummy, remap=True)
        g = [pltpu.async_copy(xs_hbm.at[src_r[r]],
                              rows.at[pl.ds(r * 128, 128)], gsem)
             for r in range(CHUNK_ROWS)]
        s = []
        for r in range(CHUNK_ROWS):
            g[r].wait()
            s.append(pltpu.async_copy(rows.at[pl.ds(r * 128, 128)],
                                      acc.at[dst_r[r]], ssem, add=True))
        for h in s:
            h.wait()
        return carry

    lax.fori_loop(0, N_CHUNKS, chunk_body, 0)
    plsc.subcore_barrier()
    _sc_copy_out(acc, out_hbm, sid, node_base)


@functools.partial(
    pl.kernel,
    mesh=_sc_mesh,
    out_type=jax.ShapeDtypeStruct((N_NODES, DP), jnp.float32),
    scratch_types=(
        [pltpu.VMEM((CHUNK_ROWS * 128,), jnp.int32)]
        + [pltpu.VMEM((128,), jnp.int32)] * CHUNK_ROWS
        + [
            pltpu.VMEM((128, DP), jnp.float32),
            pltpu.VMEM_SHARED((ACC_ROWS, DP), jnp.float32),
            pltpu.SemaphoreType.DMA,
        ]
    ),
    compiler_params=_sc_params,
)
def _sc_degree(dst_hbm, ones_hbm, zeros_hbm, out_hbm, *scratch):
    dstv = scratch[0]
    dst_r = scratch[1:1 + CHUNK_ROWS]
    ones_v, acc, ssem = scratch[1 + CHUNK_ROWS:]

    cid = lax.axis_index("c")
    sid = lax.axis_index("s")
    node_base = cid * HALF
    dummy = HALF + sid

    pltpu.sync_copy(ones_hbm, ones_v)
    _sc_zero_acc(zeros_hbm, acc, sid)
    plsc.subcore_barrier()

    def chunk_body(c, carry):
        eb = (sid * EROWS_PER_TILE + c * CHUNK_ROWS) * 128
        pltpu.sync_copy(dst_hbm.at[pl.ds(eb, CHUNK_ROWS * 128)], dstv)
        _remap_to_1d(dstv, dst_r, node_base, dummy, remap=True)
        s = [pltpu.async_copy(ones_v, acc.at[dst_r[r]], ssem, add=True)
             for r in range(CHUNK_ROWS)]
        for h in s:
            h.wait()
        return carry

    lax.fori_loop(0, N_CHUNKS, chunk_body, 0)
    plsc.subcore_barrier()
    _sc_copy_out(acc, out_hbm, sid, node_base)


def _pre_body(cnt_ref, x_ref, dinv_ref, xs_ref):
    d = lax.rsqrt(cnt_ref[...][:, :1] + 1.0)
    dinv_ref[...] = d
    xs_ref[...] = jnp.concatenate(
        [(x_ref[...] * d).astype(jnp.bfloat16),
         jnp.zeros((BLK, FP - FL), jnp.bfloat16)], axis=1)


def _tc_pre(counts, x):
    grid = (N_NODES // BLK,)
    return pl.pallas_call(
        _pre_body,
        grid=grid,
        in_specs=[
            pl.BlockSpec((BLK, DP), lambda i: (i, 0)),
            pl.BlockSpec((BLK, FL), lambda i: (i, 0)),
        ],
        out_specs=[
            pl.BlockSpec((BLK, 1), lambda i: (i, 0)),
            pl.BlockSpec((BLK, FP), lambda i: (i, 0)),
        ],
        out_shape=[
            jax.ShapeDtypeStruct((N_NODES, 1), jnp.float32),
            jax.ShapeDtypeStruct((N_NODES, FP), jnp.bfloat16),
        ],
    )(counts, x)


def _layer_body(split, agg_ref, xs_ref, dinv_ref, w_ref, b_ref, *out_refs):
    d = dinv_ref[...]
    t = (agg_ref[...].astype(jnp.float32)
         + xs_ref[...].astype(jnp.float32)) * d
    h = jnp.dot(t, w_ref[...], preferred_element_type=jnp.float32)
    z = (jnp.maximum(h + b_ref[...][None, :], 0.0) * d).astype(jnp.bfloat16)
    zpad = jnp.zeros((BLK, FP - FL), jnp.bfloat16)
    if split:
        out_refs[0][...] = jnp.concatenate([z[:, :FL], zpad], axis=1)
        out_refs[1][...] = jnp.concatenate([z[:, FL:], zpad], axis=1)
    else:
        out_refs[0][...] = jnp.concatenate([z, zpad], axis=1)


def _tc_layer(agg, xs, dinv, W, b, split):
    grid = (N_NODES // BLK,)
    n_out = 2 if split else 1
    return pl.pallas_call(
        functools.partial(_layer_body, split),
        grid=grid,
        in_specs=[
            pl.BlockSpec((BLK, FP), lambda i: (i, 0)),
            pl.BlockSpec((BLK, FP), lambda i: (i, 0)),
            pl.BlockSpec((BLK, 1), lambda i: (i, 0)),
            pl.BlockSpec(W.shape, lambda i: (0, 0)),
            pl.BlockSpec(b.shape, lambda i: (0,)),
        ],
        out_specs=[pl.BlockSpec((BLK, FP), lambda i: (i, 0))] * n_out,
        out_shape=[jax.ShapeDtypeStruct((N_NODES, FP), jnp.bfloat16)] * n_out,
    )(agg, xs, dinv, W, b)


def _head_body(a3a_ref, a3b_ref, x3a_ref, x3b_ref, dinv_ref,
               w3_ref, b3_ref, f1w_ref, f1b_ref, f2w_ref, f2b_ref, o_ref):
    d = dinv_ref[...]
    t = jnp.concatenate(
        [(a3a_ref[...].astype(jnp.float32)
          + x3a_ref[...].astype(jnp.float32)) * d,
         (a3b_ref[...].astype(jnp.float32)
          + x3b_ref[...].astype(jnp.float32)) * d], axis=1)
    h3 = jnp.maximum(
        jnp.dot(t, w3_ref[...], preferred_element_type=jnp.float32)
        + b3_ref[...][None, :], 0.0)
    z = jnp.maximum(
        jnp.dot(h3, f1w_ref[...], preferred_element_type=jnp.float32)
        + f1b_ref[...][None, :], 0.0)
    y = jnp.dot(z, f2w_ref[...], preferred_element_type=jnp.float32) \
        + f2b_ref[...][None, :]
    o_ref[...] = jax.nn.softmax(y, axis=-1)


def _tc_head(a3a, a3b, x3a, x3b, dinv, W3, b3, fc1_W, fc1_b, fc2_W, fc2_b):
    grid = (N_NODES // BLK,)
    row = lambda i: (i, 0)
    rep2 = lambda i: (0, 0)
    rep1 = lambda i: (0,)
    return pl.pallas_call(
        _head_body,
        grid=grid,
        in_specs=[
            pl.BlockSpec((BLK, FP), row),
            pl.BlockSpec((BLK, FP), row),
            pl.BlockSpec((BLK, FP), row),
            pl.BlockSpec((BLK, FP), row),
            pl.BlockSpec((BLK, 1), row),
            pl.BlockSpec(W3.shape, rep2),
            pl.BlockSpec(b3.shape, rep1),
            pl.BlockSpec(fc1_W.shape, rep2),
            pl.BlockSpec(fc1_b.shape, rep1),
            pl.BlockSpec(fc2_W.shape, rep2),
            pl.BlockSpec(fc2_b.shape, rep1),
        ],
        out_specs=pl.BlockSpec((BLK, 1), row),
        out_shape=jax.ShapeDtypeStruct((N_NODES, 1), jnp.float32),
    )(a3a, a3b, x3a, x3b, dinv, W3, b3, fc1_W, fc1_b, fc2_W, fc2_b)


def _pad_rows(W, rows):
    return jnp.concatenate(
        [W, jnp.zeros((rows - W.shape[0], W.shape[1]), W.dtype)], axis=0)


def kernel(x, edge_index, W1, b1, W2, b2, W3, b3, fc1_W, fc1_b, fc2_W, fc2_b):
    src = edge_index[0].astype(jnp.int32)
    dst = edge_index[1].astype(jnp.int32)
    pad = E_PAD - N_EDGES
    src_p = jnp.concatenate([src, jnp.zeros((pad,), jnp.int32)])
    dst_p = jnp.concatenate([dst, jnp.full((pad,), jnp.int32(2 ** 30))])
    zeros48 = jnp.zeros((ZROWS, FP), jnp.bfloat16)
    zeros8 = jnp.zeros((ZROWS, DP), jnp.float32)
    ones8 = jnp.ones((128, DP), jnp.float32)

    W1p = _pad_rows(W1, FP)
    W2p = _pad_rows(W2, FP)
    W3p = jnp.concatenate([
        _pad_rows(W3[:FL], FP), _pad_rows(W3[FL:], FP)], axis=0)

    counts = _sc_degree(dst_p, ones8, zeros8)
    dinv, xs1 = _tc_pre(counts, x)
    agg1 = _sc_scatter(src_p, dst_p, xs1, zeros48)
    (xs2,) = _tc_layer(agg1, xs1, dinv, W1p, b1, split=False)
    agg2 = _sc_scatter(src_p, dst_p, xs2, zeros48)
    xs3a, xs3b = _tc_layer(agg2, xs2, dinv, W2p, b2, split=True)
    agg3a = _sc_scatter(src_p, dst_p, xs3a, zeros48)
    agg3b = _sc_scatter(src_p, dst_p, xs3b, zeros48)
    return _tc_head(agg3a, agg3b, xs3a, xs3b, dinv,
                    W3p, b3, fc1_W, fc1_b, fc2_W, fc2_b)
